# baseline (speedup 1.0000x reference)
import functools

import jax
import jax.numpy as jnp
from jax.experimental import pallas as pl
from jax.experimental.pallas import tpu as pltpu

K_ORDER = 25


def _cheb_kernel(L_ref, X_ref, out_ref, t1_ref, t2_ref):
    k = pl.program_id(0)

    @pl.when(k == 0)
    def _():
        T = X_ref[...]
        out_ref[0] = T
        t2_ref[...] = T

    @pl.when(k == 1)
    def _():
        T = jnp.dot(L_ref[...], X_ref[...], preferred_element_type=jnp.float32)
        out_ref[0] = T
        t1_ref[...] = T

    @pl.when(k >= 2)
    def _():
        T = 2.0 * jnp.dot(L_ref[...], t1_ref[...],
                          preferred_element_type=jnp.float32) - t2_ref[...]
        out_ref[0] = T
        t2_ref[...] = t1_ref[...]
        t1_ref[...] = T


def _cheb_basis(L, X):
    N, C = X.shape
    return pl.pallas_call(
        _cheb_kernel,
        grid=(K_ORDER,),
        in_specs=[
            pl.BlockSpec((N, N), lambda k: (0, 0)),
            pl.BlockSpec((N, C), lambda k: (0, 0)),
        ],
        out_specs=pl.BlockSpec((1, N, C), lambda k: (k, 0, 0)),
        out_shape=jax.ShapeDtypeStruct((K_ORDER, N, C), jnp.float32),
        scratch_shapes=[
            pltpu.VMEM((N, C), jnp.float32),
            pltpu.VMEM((N, C), jnp.float32),
        ],
    )(L, X)


def _linear_kernel(A_ref, W_ref, b_ref, out_ref, *, relu):
    h = jnp.dot(A_ref[...], W_ref[...], preferred_element_type=jnp.float32)
    h = h + b_ref[...]
    if relu:
        h = jnp.maximum(h, 0.0)
    out_ref[...] = h


def _linear_relu(A, W, b, block_m=3584):
    M, F = A.shape
    G = W.shape[1]
    grid = M // block_m
    return pl.pallas_call(
        functools.partial(_linear_kernel, relu=True),
        grid=(grid,),
        in_specs=[
            pl.BlockSpec((block_m, F), lambda i: (i, 0)),
            pl.BlockSpec((F, G), lambda i: (0, 0)),
            pl.BlockSpec((1, G), lambda i: (0, 0)),
        ],
        out_specs=pl.BlockSpec((block_m, G), lambda i: (i, 0)),
        out_shape=jax.ShapeDtypeStruct((M, G), jnp.float32),
    )(A, W, b.reshape(1, G))


def _fc_kernel(h_ref, W1_ref, b1_ref, W2_ref, b2_ref, out_ref):
    h1 = jnp.dot(h_ref[...], W1_ref[...], preferred_element_type=jnp.float32)
    h1 = jnp.maximum(h1 + b1_ref[...], 0.0)
    h2 = jnp.dot(h1, W2_ref[...], preferred_element_type=jnp.float32)
    h2 = h2 + b2_ref[...]
    m = jnp.max(h2, axis=1, keepdims=True)
    lse = jnp.log(jnp.sum(jnp.exp(h2 - m), axis=1, keepdims=True)) + m
    out_ref[...] = h2 - lse


def _fc_head(h, fc1W, fc1b, fc2W, fc2b):
    B, D = h.shape
    H1 = fc1W.shape[1]
    G = fc2W.shape[1]
    return pl.pallas_call(
        _fc_kernel,
        in_specs=[
            pl.BlockSpec((B, D), lambda: (0, 0)),
            pl.BlockSpec((D, H1), lambda: (0, 0)),
            pl.BlockSpec((1, H1), lambda: (0, 0)),
            pl.BlockSpec((H1, G), lambda: (0, 0)),
            pl.BlockSpec((1, G), lambda: (0, 0)),
        ],
        out_specs=pl.BlockSpec((B, G), lambda: (0, 0)),
        out_shape=jax.ShapeDtypeStruct((B, G), jnp.float32),
    )(h, fc1W, fc1b.reshape(1, H1), fc2W, fc2b.reshape(1, G))


def _gcn_layer(L, X, W, b, fin):
    N = X.shape[0]
    B = X.shape[1] // fin
    fout = W.shape[1]
    Xs = _cheb_basis(L, X)
    A = (Xs.reshape(K_ORDER, N, B, fin)
         .transpose(1, 2, 3, 0)
         .reshape(N * B, fin * K_ORDER))
    H = _linear_relu(A, W, b)
    return H.reshape(N, B * fout)


def kernel(x, L, W1, b1, W2, b2, W3, b3, fc1W, fc1b, fc2W, fc2b):
    B, N, _ = x.shape
    X = x[:, :, 0].T
    H1 = _gcn_layer(L, X, W1, b1, 1)
    H2 = _gcn_layer(L, H1, W2, b2, 30)
    H3 = _gcn_layer(L, H2, W3, b3, 20)
    Hf = H3.reshape(N, B, 10).transpose(1, 0, 2).reshape(B, N * 10)
    return _fc_head(Hf, fc1W, fc1b, fc2W, fc2b)
